# baseline (speedup 1.0000x reference)
import functools

import jax
import jax.numpy as jnp
from jax import lax
from jax.experimental import pallas as pl
from jax.experimental.pallas import tpu as pltpu
from jax.experimental.pallas import tpu_sc as plsc

BBLK = 128
NROW = 4
NSLAB = 3
GDEPTH = 3
TNB = 3


@functools.partial(jax.jit, static_argnames=("emb", "vocab"))
def _sc_transpose(wt, wtt, *, emb, vocab):
    mesh = plsc.VectorSubcoreMesh(core_axis_name="c", subcore_axis_name="s")
    nc = mesh.num_cores
    nw = nc * mesh.num_subcores
    nfull = vocab // 128
    ntiles = nfull + (1 if vocab % 128 else 0)
    vpad = ntiles * 128
    per_w = (ntiles + nw - 1) // nw
    n_m = (emb * 128) // 16
    em = emb // 16
    blk_words = 128 * emb

    def body(wt_hbm, wtt_hbm, out_hbm, in_v, out_v, rsem, wsem):
        wid = lax.axis_index("s") * nc + lax.axis_index("c")
        lane = lax.iota(jnp.int32, 16)

        def task(k):
            return wid * per_w + k

        def start_read(k, buf):
            t = task(k)

            @pl.when(t < nfull)
            def _():
                pltpu.make_async_copy(
                    wt_hbm.at[:, pl.ds(pl.multiple_of(t * 128, 128), 128)],
                    in_v.at[buf],
                    rsem,
                ).start()

            @pl.when(jnp.logical_and(t >= nfull, t < ntiles))
            def _():
                pltpu.make_async_copy(wtt_hbm, in_v.at[buf], rsem).start()

        def wait_read(buf):
            pltpu.make_async_copy(
                wt_hbm.at[:, pl.ds(0, 128)], in_v.at[buf], rsem
            ).wait()

        def write_desc(k, ob):
            return pltpu.make_async_copy(
                out_v.at[pl.ds(ob * blk_words, blk_words)],
                out_hbm.at[pl.ds(task(k) * blk_words, blk_words)],
                wsem,
            )

        def permute(buf, ob):
            obase = ob * blk_words

            @plsc.parallel_loop(0, n_m, unroll=8)
            def _(m):
                e0 = lax.rem(m, em) * 16
                vl = lax.div(m, em)
                v = plsc.load_gather(
                    in_v.at[buf], [e0 + lane, jnp.full((16,), vl, jnp.int32)]
                )
                out_v[pl.ds(obase + m * 16, 16)] = v

        for p in range(min(TNB, per_w)):
            start_read(p, p)

        @pl.loop(0, per_w)
        def _(k):
            t = task(k)

            @pl.when(t < ntiles)
            def _():
                b = lax.rem(k, TNB)
                wait_read(b)

                @pl.when(k >= TNB)
                def _():
                    write_desc(k - TNB, lax.rem(k - TNB, TNB)).wait()

                permute(b, b)
                write_desc(k, b).start()

            kn = k + TNB

            @pl.when(kn < per_w)
            def _():
                start_read(kn, lax.rem(kn, TNB))

        nvalid = jnp.clip(ntiles - wid * per_w, 0, per_w)

        @pl.loop(0, TNB)
        def _(q):
            kk = nvalid - TNB + q

            @pl.when(kk >= 0)
            def _():
                write_desc(kk, lax.rem(kk, TNB)).wait()

    run = pl.kernel(
        body,
        out_type=jax.ShapeDtypeStruct((vpad * emb,), jnp.float32),
        mesh=mesh,
        compiler_params=pltpu.CompilerParams(
            use_tc_tiling_on_sc=True, needs_layout_passes=False
        ),
        scratch_types=[
            pltpu.VMEM((TNB, emb, 128), jnp.float32),
            pltpu.VMEM((TNB * blk_words,), jnp.float32),
            pltpu.SemaphoreType.DMA,
            pltpu.SemaphoreType.DMA,
        ],
    )
    return run(wt, wtt)


@functools.partial(jax.jit, static_argnames=("length", "emb", "ntc"))
def _sc_gather(idx2, table, *, length, emb, ntc):
    mesh = plsc.VectorSubcoreMesh(core_axis_name="c", subcore_axis_name="s")
    nc = mesh.num_cores
    nw = nc * mesh.num_subcores
    ntasks = length * ntc
    per_w = ntasks // nw
    eh = emb // 8
    n_m = (emb * BBLK) // 16
    bm = BBLK // 16

    def body(table_hbm, idx_hbm, out_hbm, idx_v, rows_v, slab_v, gsem, wsem):
        wid = lax.axis_index("s") * nc + lax.axis_index("c")
        t0 = wid * per_w
        pltpu.sync_copy(idx_hbm.at[pl.ds(t0, per_w)], idx_v)

        def gather_desc(j, buf):
            return pltpu.make_async_copy(
                table_hbm.at[idx_v.at[j]], rows_v.at[buf], gsem
            )

        def write_desc(j, sb):
            t = t0 + j
            return pltpu.make_async_copy(
                slab_v.at[sb],
                out_hbm.at[lax.div(t, ntc), :, lax.rem(t, ntc)],
                wsem,
            )

        lane = lax.iota(jnp.int32, 16)

        def permute(buf, sb):
            @plsc.parallel_loop(0, n_m, unroll=8)
            def _(m):
                e = lax.div(m, bm)
                b0 = lax.rem(m, bm) * 16
                v = plsc.load_gather(
                    rows_v.at[buf], [b0 + lane, jnp.full((16,), e, jnp.int32)]
                )
                slab_v[sb, lax.div(e, 8), lax.rem(e, 8), pl.ds(b0, 16)] = v

        for p in range(GDEPTH):
            gather_desc(p, p).start()

        @pl.loop(0, per_w)
        def _(j):
            gather_desc(j, lax.rem(j, NROW)).wait()

            @pl.when(j >= NSLAB)
            def _():
                write_desc(j - NSLAB, lax.rem(j - NSLAB, NSLAB)).wait()

            sb = lax.rem(j, NSLAB)
            permute(lax.rem(j, NROW), sb)
            write_desc(j, sb).start()

            @pl.when(j + GDEPTH < per_w)
            def _():
                jn = j + GDEPTH
                gather_desc(jn, lax.rem(jn, NROW)).start()

        @pl.loop(0, NSLAB)
        def _(t):
            jj = per_w - NSLAB + t
            write_desc(jj, lax.rem(jj, NSLAB)).wait()

    run = pl.kernel(
        body,
        out_type=jax.ShapeDtypeStruct((length, eh, ntc, 8, BBLK), jnp.float32),
        mesh=mesh,
        compiler_params=pltpu.CompilerParams(
            use_tc_tiling_on_sc=False, needs_layout_passes=False
        ),
        scratch_types=[
            pltpu.VMEM((per_w, BBLK), jnp.int32),
            pltpu.VMEM((NROW, BBLK, emb), jnp.float32),
            pltpu.VMEM((NSLAB, eh, 8, BBLK), jnp.float32),
            pltpu.SemaphoreType.DMA,
            pltpu.SemaphoreType.DMA,
        ],
    )
    return run(table, idx2)


def kernel(inputs, shared_weights):
    bsz, length = inputs.shape
    vocab, emb = shared_weights.shape
    ntc = bsz // BBLK
    assert ntc * BBLK == bsz and emb % 16 == 0
    idx = inputs if inputs.dtype == jnp.int32 else inputs.astype(jnp.int32)
    idx2 = idx.T.reshape(length * ntc, BBLK)
    wt = shared_weights.T
    ntail = vocab % 128
    tail = wt[:, vocab - ntail:] if ntail else wt[:, :0]
    wtt = jnp.pad(tail, ((0, 0), (0, 128 - ntail)))
    vpad = vocab + ((128 - ntail) % 128)
    t_lin = _sc_transpose(wt, wtt, emb=emb, vocab=vocab)
    t2 = t_lin.reshape(vpad, emb)
    out5 = _sc_gather(idx2, t2, length=length, emb=emb, ntc=ntc)
    return out5.transpose(2, 4, 0, 1, 3).reshape(bsz, length, emb)


# speedup vs baseline: 1.3315x; 1.3315x over previous
import functools

import jax
import jax.numpy as jnp
from jax import lax
from jax.experimental import pallas as pl
from jax.experimental.pallas import tpu as pltpu
from jax.experimental.pallas import tpu_sc as plsc

BBLK = 128
NROW = 4
NSLAB = 3
GDEPTH = 3
TNB = 3


@functools.partial(jax.jit, static_argnames=("emb", "vocab"))
def _sc_transpose(wt, wtt, *, emb, vocab):
    mesh = plsc.VectorSubcoreMesh(core_axis_name="c", subcore_axis_name="s")
    nc = mesh.num_cores
    nw = nc * mesh.num_subcores
    nfull = vocab // 128
    ntiles = nfull + (1 if vocab % 128 else 0)
    vpad = ntiles * 128
    per_w = (ntiles + nw - 1) // nw
    n_m = (emb * 128) // 16
    em = emb // 16
    blk_words = 128 * emb

    def body(wt_hbm, wtt_hbm, out_hbm, in_v, out_v, rsem, wsem):
        wid = lax.axis_index("s") * nc + lax.axis_index("c")
        lane = lax.iota(jnp.int32, 16)

        def task(k):
            return wid * per_w + k

        def start_read(k, buf):
            t = task(k)

            @pl.when(t < nfull)
            def _():
                pltpu.make_async_copy(
                    wt_hbm.at[:, pl.ds(pl.multiple_of(t * 128, 128), 128)],
                    in_v.at[buf],
                    rsem,
                ).start()

            @pl.when(jnp.logical_and(t >= nfull, t < ntiles))
            def _():
                pltpu.make_async_copy(wtt_hbm, in_v.at[buf], rsem).start()

        def wait_read(buf):
            pltpu.make_async_copy(
                wt_hbm.at[:, pl.ds(0, 128)], in_v.at[buf], rsem
            ).wait()

        def write_desc(k, ob):
            return pltpu.make_async_copy(
                out_v.at[pl.ds(ob * blk_words, blk_words)],
                out_hbm.at[pl.ds(task(k) * blk_words, blk_words)],
                wsem,
            )

        lanes = [lane + 16 * q for q in range(em)]

        def permute(buf, ob):
            obase = ob * blk_words

            @plsc.parallel_loop(0, 128, unroll=4)
            def _(vl):
                vs = jnp.full((16,), vl, jnp.int32)
                vbase = obase + vl * emb
                for q in range(em):
                    v = plsc.load_gather(in_v.at[buf], [lanes[q], vs])
                    out_v[pl.ds(vbase + 16 * q, 16)] = v

        for p in range(min(TNB, per_w)):
            start_read(p, p)

        @pl.loop(0, per_w)
        def _(k):
            t = task(k)

            @pl.when(t < ntiles)
            def _():
                b = lax.rem(k, TNB)
                wait_read(b)

                @pl.when(k >= TNB)
                def _():
                    write_desc(k - TNB, lax.rem(k - TNB, TNB)).wait()

                permute(b, b)
                write_desc(k, b).start()

            kn = k + TNB

            @pl.when(kn < per_w)
            def _():
                start_read(kn, lax.rem(kn, TNB))

        nvalid = jnp.clip(ntiles - wid * per_w, 0, per_w)

        @pl.loop(0, TNB)
        def _(q):
            kk = nvalid - TNB + q

            @pl.when(kk >= 0)
            def _():
                write_desc(kk, lax.rem(kk, TNB)).wait()

    run = pl.kernel(
        body,
        out_type=jax.ShapeDtypeStruct((vpad * emb,), jnp.float32),
        mesh=mesh,
        compiler_params=pltpu.CompilerParams(
            use_tc_tiling_on_sc=True, needs_layout_passes=False
        ),
        scratch_types=[
            pltpu.VMEM((TNB, emb, 128), jnp.float32),
            pltpu.VMEM((TNB * blk_words,), jnp.float32),
            pltpu.SemaphoreType.DMA,
            pltpu.SemaphoreType.DMA,
        ],
    )
    return run(wt, wtt)


@functools.partial(jax.jit, static_argnames=("length", "emb", "ntc"))
def _sc_gather(idx2, table, *, length, emb, ntc):
    mesh = plsc.VectorSubcoreMesh(core_axis_name="c", subcore_axis_name="s")
    nc = mesh.num_cores
    nw = nc * mesh.num_subcores
    ntasks = length * ntc
    per_w = ntasks // nw
    eh = emb // 8
    n_m = (emb * BBLK) // 16
    bm = BBLK // 16

    def body(table_hbm, idx_hbm, out_hbm, idx_v, rows_v, slab_v, gsem, wsem):
        wid = lax.axis_index("s") * nc + lax.axis_index("c")
        t0 = wid * per_w
        pltpu.sync_copy(idx_hbm.at[pl.ds(t0, per_w)], idx_v)

        def gather_desc(j, buf):
            return pltpu.make_async_copy(
                table_hbm.at[idx_v.at[j]], rows_v.at[buf], gsem
            )

        def write_desc(j, sb):
            t = t0 + j
            return pltpu.make_async_copy(
                slab_v.at[sb],
                out_hbm.at[lax.div(t, ntc), :, lax.rem(t, ntc)],
                wsem,
            )

        lane = lax.iota(jnp.int32, 16)
        blanes = [lane + 16 * q for q in range(bm)]

        def permute(buf, sb):
            @plsc.parallel_loop(0, emb, unroll=4)
            def _(e):
                ehi = lax.div(e, 8)
                elo = lax.rem(e, 8)
                es = jnp.full((16,), e, jnp.int32)
                for q in range(bm):
                    v = plsc.load_gather(rows_v.at[buf], [blanes[q], es])
                    slab_v[sb, ehi, elo, pl.ds(16 * q, 16)] = v

        for p in range(GDEPTH):
            gather_desc(p, p).start()

        @pl.loop(0, per_w)
        def _(j):
            gather_desc(j, lax.rem(j, NROW)).wait()

            @pl.when(j >= NSLAB)
            def _():
                write_desc(j - NSLAB, lax.rem(j - NSLAB, NSLAB)).wait()

            sb = lax.rem(j, NSLAB)
            permute(lax.rem(j, NROW), sb)
            write_desc(j, sb).start()

            @pl.when(j + GDEPTH < per_w)
            def _():
                jn = j + GDEPTH
                gather_desc(jn, lax.rem(jn, NROW)).start()

        @pl.loop(0, NSLAB)
        def _(t):
            jj = per_w - NSLAB + t
            write_desc(jj, lax.rem(jj, NSLAB)).wait()

    run = pl.kernel(
        body,
        out_type=jax.ShapeDtypeStruct((length, eh, ntc, 8, BBLK), jnp.float32),
        mesh=mesh,
        compiler_params=pltpu.CompilerParams(
            use_tc_tiling_on_sc=False, needs_layout_passes=False
        ),
        scratch_types=[
            pltpu.VMEM((per_w, BBLK), jnp.int32),
            pltpu.VMEM((NROW, BBLK, emb), jnp.float32),
            pltpu.VMEM((NSLAB, eh, 8, BBLK), jnp.float32),
            pltpu.SemaphoreType.DMA,
            pltpu.SemaphoreType.DMA,
        ],
    )
    return run(table, idx2)


def kernel(inputs, shared_weights):
    bsz, length = inputs.shape
    vocab, emb = shared_weights.shape
    ntc = bsz // BBLK
    assert ntc * BBLK == bsz and emb % 16 == 0
    idx = inputs if inputs.dtype == jnp.int32 else inputs.astype(jnp.int32)
    idx2 = idx.T.reshape(length * ntc, BBLK)
    wt = shared_weights.T
    ntail = vocab % 128
    tail = wt[:, vocab - ntail:] if ntail else wt[:, :0]
    wtt = jnp.pad(tail, ((0, 0), (0, 128 - ntail)))
    vpad = vocab + ((128 - ntail) % 128)
    t_lin = _sc_transpose(wt, wtt, emb=emb, vocab=vocab)
    t2 = t_lin.reshape(vpad, emb)
    out5 = _sc_gather(idx2, t2, length=length, emb=emb, ntc=ntc)
    return out5.transpose(2, 4, 0, 1, 3).reshape(bsz, length, emb)


# speedup vs baseline: 1.5635x; 1.1742x over previous
import functools

import jax
import jax.numpy as jnp
from jax import lax
from jax.experimental import pallas as pl
from jax.experimental.pallas import tpu as pltpu
from jax.experimental.pallas import tpu_sc as plsc

NBUF = 6
GAHEAD = 4


@functools.partial(jax.jit, static_argnames=("rows_per_w", "length", "emb"))
def _sc_gather(idx, table, *, rows_per_w, length, emb):
    bsz = idx.shape[0]
    mesh = plsc.VectorSubcoreMesh(core_axis_name="c", subcore_axis_name="s")
    nc = mesh.num_cores
    c0 = 128 if length > 128 else length
    c1 = length - c0

    def body(table_hbm, idx_hbm, out_hbm, idx_v, rows_v, gsem, wsem):
        wid = lax.axis_index("s") * nc + lax.axis_index("c")
        base = wid * rows_per_w
        pltpu.sync_copy(idx_hbm.at[pl.ds(base, rows_per_w)], idx_v)

        def fire_gather(row, buf):
            pltpu.async_copy(
                table_hbm.at[idx_v.at[row, pl.ds(0, c0)]],
                rows_v.at[buf, pl.ds(0, c0)],
                gsem,
            )
            if c1:
                pltpu.async_copy(
                    table_hbm.at[idx_v.at[row, pl.ds(c0, c1)]],
                    rows_v.at[buf, pl.ds(c0, c1)],
                    gsem,
                )

        def wait_gather(row, buf):
            pltpu.make_async_copy(
                table_hbm.at[idx_v.at[row, pl.ds(0, c0)]],
                rows_v.at[buf, pl.ds(0, c0)],
                gsem,
            ).wait()
            if c1:
                pltpu.make_async_copy(
                    table_hbm.at[idx_v.at[row, pl.ds(c0, c1)]],
                    rows_v.at[buf, pl.ds(c0, c1)],
                    gsem,
                ).wait()

        def write_desc(row, buf):
            return pltpu.make_async_copy(
                rows_v.at[buf], out_hbm.at[base + row], wsem
            )

        @pl.loop(0, rows_per_w + GAHEAD)
        def _(j):
            @pl.when(j < rows_per_w)
            def _():
                b = lax.rem(j, NBUF)

                @pl.when(j >= NBUF)
                def _():
                    write_desc(j - NBUF, b).wait()

                fire_gather(j, b)

            @pl.when(j >= GAHEAD)
            def _():
                jj = j - GAHEAD
                bb = lax.rem(jj, NBUF)
                wait_gather(jj, bb)
                write_desc(jj, bb).start()

        @pl.loop(0, NBUF)
        def _(t):
            jj = rows_per_w - NBUF + t
            write_desc(jj, lax.rem(jj, NBUF)).wait()

    run = pl.kernel(
        body,
        out_type=jax.ShapeDtypeStruct((bsz, length, emb), jnp.float32),
        mesh=mesh,
        compiler_params=pltpu.CompilerParams(use_tc_tiling_on_sc=False),
        scratch_types=[
            pltpu.VMEM((rows_per_w, length), jnp.int32),
            pltpu.VMEM((NBUF, length, emb), jnp.float32),
            pltpu.SemaphoreType.DMA,
            pltpu.SemaphoreType.DMA,
        ],
    )
    return run(table, idx)


def kernel(inputs, shared_weights):
    bsz, length = inputs.shape
    vocab, emb = shared_weights.shape
    info = plsc.get_sparse_core_info()
    n_workers = info.num_cores * info.num_subcores
    rows_per_w = bsz // n_workers
    assert rows_per_w * n_workers == bsz
    idx = inputs if inputs.dtype == jnp.int32 else inputs.astype(jnp.int32)
    return _sc_gather(
        idx, shared_weights, rows_per_w=rows_per_w, length=length, emb=emb
    )
